# baseline (speedup 1.0000x reference)
import functools

import jax
import jax.numpy as jnp
from jax import lax
from jax.experimental import pallas as pl
from jax.experimental.pallas import tpu as pltpu
from jax.experimental.pallas import tpu_sc as plsc

_NB = 256
_N = 1024
_K = 256
_NW = 32
_CPW = _N // _NW
_L = 16
_PAD = _N + _K


def _tc_kernel(wr_ref, idr_ref, wcb_ref, idcb_ref, res_ref, out_ref):
    i = pl.program_id(1)

    wr = wr_ref[...]
    idr = idr_ref[...]
    logits_r = jnp.where(idr == 1, wr, -1e30)
    m = jnp.max(logits_r)
    er = jnp.exp(logits_r - m)
    denom = jnp.sum(er)

    @pl.when(i == 0)
    def _init():
        out_ref[...] = jnp.zeros_like(out_ref)

    blk = res_ref[...]
    lg_blk = jnp.where(idcb_ref[...] == 1, wcb_ref[...], -1e30)
    w_blk = jnp.exp(lg_blk - m) / denom
    out_ref[...] += jnp.sum(blk * w_blk[:, :, None], axis=0)


def _sc_topk(w_hbm, id_hbm, pc_hbm, pen_hbm, vals_hbm, idxs_hbm,
             w_v, id_v, pc_v, pen_v, sc_v, myv_v, myr_v, myi_v, sem):
    wid = lax.axis_index("s") * 2 + lax.axis_index("c")
    gbase = wid * _CPW

    pltpu.sync_copy(w_hbm, w_v)
    pltpu.sync_copy(id_hbm, id_v)
    pltpu.sync_copy(pc_hbm, pc_v)
    pltpu.sync_copy(pen_hbm, pen_v)
    pen = pen_v[...]

    nvec = _N // _L

    def _mx(v, acc):
        lg = jnp.where(id_v[pl.ds(v * _L, _L)] == 1,
                       w_v[pl.ds(v * _L, _L)], -1e30)
        return jnp.maximum(acc, lg)
    m = jnp.max(lax.fori_loop(0, nvec, _mx,
                              jnp.full((_L,), -1e30, jnp.float32)))

    def _sm(v, acc):
        lg = jnp.where(id_v[pl.ds(v * _L, _L)] == 1,
                       w_v[pl.ds(v * _L, _L)], -1e30)
        return acc + jnp.exp(lg - m)
    denom = jnp.sum(lax.fori_loop(0, nvec, _sm,
                                  jnp.zeros((_L,), jnp.float32)))

    def _sc(v, _):
        msk = id_v[pl.ds(v * _L, _L)] == 1
        lg = jnp.where(msk, w_v[pl.ds(v * _L, _L)], -1e30)
        s = jnp.exp(lg - m) / denom - pen * pc_v[pl.ds(v * _L, _L)]
        sc_v[pl.ds(v * _L, _L)] = jnp.where(msk, s, -jnp.inf)
        return 0
    lax.fori_loop(0, nvec, _sc, 0)

    lane = lax.iota(jnp.int32, _L)
    for g in range(_CPW // _L):
        grp = gbase + g * _L
        grp_vec = sc_v[pl.ds(grp, _L)]

        def _cand(c, rankvec):
            ig = grp + c
            ssc = jnp.sum(jnp.where(lane == c, grp_vec, 0.0))
            sv = jnp.zeros((_L,), jnp.float32) + ssc

            def _cnt(v, acc):
                a = acc
                for k in range(8):
                    off = v * (_L * 8) + k * _L
                    u = sc_v[pl.ds(off, _L)]
                    vidx = lane + off
                    beat = (u > sv) | ((u == sv) & (vidx < ig))
                    a = a + beat.astype(jnp.int32)
                return a
            acc = lax.fori_loop(0, nvec // 8, _cnt,
                                jnp.zeros((_L,), jnp.int32))
            rank = jnp.sum(acc)
            row = g * _L + c
            myv_v[row, pl.ds(0, _L)] = sv
            myi_v[row, pl.ds(0, _L)] = jnp.full((_L,), 0, jnp.int32) + ig
            return jnp.where(lane == c, rank, rankvec)

        rankvec = lax.fori_loop(0, _L, _cand,
                                jnp.zeros((_L,), jnp.int32))
        idxvec = lane + grp
        slot = jnp.where(rankvec < _K, rankvec, _K + idxvec)
        myr_v[pl.ds(g * _L, _L)] = slot

    pltpu.async_copy(myv_v, vals_hbm.at[myr_v], sem).wait()
    pltpu.async_copy(myi_v, idxs_hbm.at[myr_v], sem).wait()


def kernel(W, W_id, results, prog_cost, penalty, topN):
    N, B, D = results.shape
    K = _K
    Dc = D // 2
    wr = W.reshape(1, N)
    idr = W_id.reshape(1, N)
    wc = W.reshape(N, 1)
    idc = W_id.reshape(N, 1)
    pen16 = jnp.broadcast_to(penalty, (_L,))

    sc_call = functools.partial(
        pl.kernel,
        mesh=plsc.VectorSubcoreMesh(core_axis_name="c", subcore_axis_name="s"),
        compiler_params=pltpu.CompilerParams(needs_layout_passes=False),
        out_type=[
            jax.ShapeDtypeStruct((_PAD, 128), jnp.float32),
            jax.ShapeDtypeStruct((_PAD, 128), jnp.int32),
        ],
        scratch_types=[
            pltpu.VMEM((N,), jnp.float32),
            pltpu.VMEM((N,), jnp.int32),
            pltpu.VMEM((N,), jnp.float32),
            pltpu.VMEM((_L,), jnp.float32),
            pltpu.VMEM((N,), jnp.float32),
            pltpu.VMEM((_CPW, 128), jnp.float32),
            pltpu.VMEM((_CPW,), jnp.int32),
            pltpu.VMEM((_CPW, 128), jnp.int32),
            pltpu.SemaphoreType.DMA,
        ],
    )(_sc_topk)
    neww_pad, topi_pad = sc_call(W, W_id, prog_cost, pen16)

    grid = (2, N // _NB)
    out = pl.pallas_call(
        _tc_kernel,
        grid=grid,
        in_specs=[
            pl.BlockSpec((1, N), lambda j, i: (0, 0)),
            pl.BlockSpec((1, N), lambda j, i: (0, 0)),
            pl.BlockSpec((_NB, 1), lambda j, i: (i, 0)),
            pl.BlockSpec((_NB, 1), lambda j, i: (i, 0)),
            pl.BlockSpec((_NB, B, Dc), lambda j, i: (i, 0, j)),
        ],
        out_specs=pl.BlockSpec((B, Dc), lambda j, i: (0, j)),
        out_shape=jax.ShapeDtypeStruct((B, D), jnp.float32),
        compiler_params=pltpu.CompilerParams(
            dimension_semantics=("parallel", "arbitrary"),
        ),
    )(wr, idr, wc, idc, results)
    return out, neww_pad[:K, 0], topi_pad[:K, 0]


# speedup vs baseline: 1.0055x; 1.0055x over previous
import functools

import jax
import jax.numpy as jnp
from jax import lax
from jax.experimental import pallas as pl
from jax.experimental.pallas import tpu as pltpu
from jax.experimental.pallas import tpu_sc as plsc

_NB = 256
_N = 1024
_K = 256
_NW = 32
_CPW = _N // _NW
_L = 16
_PAD = _N + _K


def _tc_kernel(wr_ref, idr_ref, wcb_ref, idcb_ref, res_ref, out_ref):
    i = pl.program_id(1)

    wr = wr_ref[...]
    idr = idr_ref[...]
    logits_r = jnp.where(idr == 1, wr, -1e30)
    m = jnp.max(logits_r)
    er = jnp.exp(logits_r - m)
    denom = jnp.sum(er)

    @pl.when(i == 0)
    def _init():
        out_ref[...] = jnp.zeros_like(out_ref)

    blk = res_ref[...]
    lg_blk = jnp.where(idcb_ref[...] == 1, wcb_ref[...], -1e30)
    w_blk = jnp.exp(lg_blk - m) / denom
    out_ref[...] += jnp.sum(blk * w_blk[:, :, None], axis=0)


def _sc_topk(w_hbm, id_hbm, pc_hbm, pen_hbm, vals_hbm, idxs_hbm,
             w_v, id_v, pc_v, pen_v, sc_v, myv_v, myr_v, myi_v, sem):
    wid = lax.axis_index("s") * 2 + lax.axis_index("c")
    gbase = wid * _CPW

    pltpu.sync_copy(w_hbm, w_v)
    pltpu.sync_copy(id_hbm, id_v)
    pltpu.sync_copy(pc_hbm, pc_v)
    pltpu.sync_copy(pen_hbm, pen_v)
    pen = pen_v[...]

    nvec = _N // _L

    def _mx(v, acc):
        lg = jnp.where(id_v[pl.ds(v * _L, _L)] == 1,
                       w_v[pl.ds(v * _L, _L)], -1e30)
        return jnp.maximum(acc, lg)
    m = jnp.max(lax.fori_loop(0, nvec, _mx,
                              jnp.full((_L,), -1e30, jnp.float32)))

    def _sm(v, acc):
        lg = jnp.where(id_v[pl.ds(v * _L, _L)] == 1,
                       w_v[pl.ds(v * _L, _L)], -1e30)
        return acc + jnp.exp(lg - m)
    denom = jnp.sum(lax.fori_loop(0, nvec, _sm,
                                  jnp.zeros((_L,), jnp.float32)))

    def _sc(v, _):
        msk = id_v[pl.ds(v * _L, _L)] == 1
        lg = jnp.where(msk, w_v[pl.ds(v * _L, _L)], -1e30)
        s = jnp.exp(lg - m) / denom - pen * pc_v[pl.ds(v * _L, _L)]
        sc_v[pl.ds(v * _L, _L)] = jnp.where(msk, s, -jnp.inf)
        return 0
    lax.fori_loop(0, nvec, _sc, 0)

    lane = lax.iota(jnp.int32, _L)
    for g in range(_CPW // _L):
        grp = gbase + g * _L
        grp_vec = sc_v[pl.ds(grp, _L)]

        def _cand(c, rankvec):
            ig = grp + c
            ssc = jnp.sum(jnp.where(lane == c, grp_vec, 0.0))
            sv = jnp.zeros((_L,), jnp.float32) + ssc

            def _cnt(v, acc):
                a = acc
                for k in range(8):
                    off = v * (_L * 8) + k * _L
                    u = sc_v[pl.ds(off, _L)]
                    vidx = lane + off
                    beat = (u > sv) | ((u == sv) & (vidx < ig))
                    a = a + beat.astype(jnp.int32)
                return a
            acc = lax.fori_loop(0, nvec // 8, _cnt,
                                jnp.zeros((_L,), jnp.int32))
            rank = jnp.sum(acc)
            row = g * _L + c
            myv_v[row, pl.ds(0, _L)] = sv
            myi_v[row, pl.ds(0, _L)] = jnp.full((_L,), 0, jnp.int32) + ig
            return jnp.where(lane == c, rank, rankvec)

        rankvec = lax.fori_loop(0, _L, _cand,
                                jnp.zeros((_L,), jnp.int32))
        idxvec = lane + grp
        slot = jnp.where(rankvec < _K, rankvec, _K + idxvec)
        myr_v[pl.ds(g * _L, _L)] = slot

    pltpu.async_copy(myv_v, vals_hbm.at[myr_v], sem).wait()
    pltpu.async_copy(myi_v, idxs_hbm.at[myr_v], sem).wait()


def kernel(W, W_id, results, prog_cost, penalty, topN):
    N, B, D = results.shape
    K = _K
    Dc = D // 2
    wr = W.reshape(1, N)
    idr = W_id.reshape(1, N)
    wc = W.reshape(N, 1)
    idc = W_id.reshape(N, 1)
    pen16 = jnp.broadcast_to(penalty, (_L,))

    sc_call = functools.partial(
        pl.kernel,
        mesh=plsc.VectorSubcoreMesh(core_axis_name="c", subcore_axis_name="s"),
        compiler_params=pltpu.CompilerParams(needs_layout_passes=False),
        out_type=[
            jax.ShapeDtypeStruct((_PAD, 128), jnp.float32),
            jax.ShapeDtypeStruct((_PAD, 128), jnp.int32),
        ],
        scratch_types=[
            pltpu.VMEM((N,), jnp.float32),
            pltpu.VMEM((N,), jnp.int32),
            pltpu.VMEM((N,), jnp.float32),
            pltpu.VMEM((_L,), jnp.float32),
            pltpu.VMEM((N,), jnp.float32),
            pltpu.VMEM((_CPW, 128), jnp.float32),
            pltpu.VMEM((_CPW,), jnp.int32),
            pltpu.VMEM((_CPW, 128), jnp.int32),
            pltpu.SemaphoreType.DMA,
        ],
    )(_sc_topk)

    grid = (2, N // _NB)
    out = pl.pallas_call(
        _tc_kernel,
        grid=grid,
        in_specs=[
            pl.BlockSpec((1, N), lambda j, i: (0, 0)),
            pl.BlockSpec((1, N), lambda j, i: (0, 0)),
            pl.BlockSpec((_NB, 1), lambda j, i: (i, 0)),
            pl.BlockSpec((_NB, 1), lambda j, i: (i, 0)),
            pl.BlockSpec((_NB, B, Dc), lambda j, i: (i, 0, j)),
        ],
        out_specs=pl.BlockSpec((B, Dc), lambda j, i: (0, j)),
        out_shape=jax.ShapeDtypeStruct((B, D), jnp.float32),
        compiler_params=pltpu.CompilerParams(
            dimension_semantics=("parallel", "arbitrary"),
        ),
    )(wr, idr, wc, idc, results)
    neww_pad, topi_pad = sc_call(W, W_id, prog_cost, pen16)
    return out, neww_pad[:K, 0], topi_pad[:K, 0]


# speedup vs baseline: 1.0085x; 1.0030x over previous
import functools

import jax
import jax.numpy as jnp
from jax import lax
from jax.experimental import pallas as pl
from jax.experimental.pallas import tpu as pltpu
from jax.experimental.pallas import tpu_sc as plsc

_NB = 256
_N = 1024
_K = 256
_NW = 32
_CPW = _N // _NW
_L = 16
_PAD = _N + _K


def _tc_kernel(wr_ref, idr_ref, wcb_ref, idcb_ref, res_ref, out_ref):
    i = pl.program_id(1)

    wr = wr_ref[...]
    idr = idr_ref[...]
    logits_r = jnp.where(idr == 1, wr, -1e30)
    m = jnp.max(logits_r)
    er = jnp.exp(logits_r - m)
    denom = jnp.sum(er)

    @pl.when(i == 0)
    def _init():
        out_ref[...] = jnp.zeros_like(out_ref)

    blk = res_ref[...]
    lg_blk = jnp.where(idcb_ref[...] == 1, wcb_ref[...], -1e30)
    w_blk = jnp.exp(lg_blk - m) / denom
    out_ref[...] += jnp.sum(blk * w_blk[:, :, None], axis=0)


def _sc_topk(w_hbm, id_hbm, pc_hbm, pen_hbm, vals_hbm, idxs_hbm,
             w_v, id_v, pc_v, pen_v, sc_v, myv_v, myr_v, myi_v, sem):
    wid = lax.axis_index("s") * 2 + lax.axis_index("c")
    gbase = wid * _CPW

    pltpu.sync_copy(w_hbm, w_v)
    pltpu.sync_copy(id_hbm, id_v)
    pltpu.sync_copy(pc_hbm, pc_v)
    pltpu.sync_copy(pen_hbm, pen_v)
    pen = pen_v[...]

    nvec = _N // _L

    def _mx(v, acc):
        lg = jnp.where(id_v[pl.ds(v * _L, _L)] == 1,
                       w_v[pl.ds(v * _L, _L)], -1e30)
        return jnp.maximum(acc, lg)
    m = jnp.max(lax.fori_loop(0, nvec, _mx,
                              jnp.full((_L,), -1e30, jnp.float32)))

    def _sm(v, acc):
        lg = jnp.where(id_v[pl.ds(v * _L, _L)] == 1,
                       w_v[pl.ds(v * _L, _L)], -1e30)
        return acc + jnp.exp(lg - m)
    denom = jnp.sum(lax.fori_loop(0, nvec, _sm,
                                  jnp.zeros((_L,), jnp.float32)))

    def _sc(v, _):
        msk = id_v[pl.ds(v * _L, _L)] == 1
        lg = jnp.where(msk, w_v[pl.ds(v * _L, _L)], -1e30)
        s = jnp.exp(lg - m) / denom - pen * pc_v[pl.ds(v * _L, _L)]
        sc_v[pl.ds(v * _L, _L)] = jnp.where(msk, s, -jnp.inf)
        return 0
    lax.fori_loop(0, nvec, _sc, 0)

    lane = lax.iota(jnp.int32, _L)
    for g in range(_CPW // _L):
        grp = gbase + g * _L
        grp_vec = sc_v[pl.ds(grp, _L)]

        def _cand(c, rankvec):
            ig = grp + c
            ssc = jnp.sum(jnp.where(lane == c, grp_vec, 0.0))
            sv = jnp.zeros((_L,), jnp.float32) + ssc

            def _cnt(v, acc):
                a = acc
                for k in range(8):
                    off = v * (_L * 8) + k * _L
                    u = sc_v[pl.ds(off, _L)]
                    vidx = lane + off
                    beat = (u > sv) | ((u == sv) & (vidx < ig))
                    a = a + beat.astype(jnp.int32)
                return a
            acc = lax.fori_loop(0, nvec // 8, _cnt,
                                jnp.zeros((_L,), jnp.int32))
            rank = jnp.sum(acc)
            row = g * _L + c
            myv_v[row, pl.ds(0, _L)] = sv
            myi_v[row, pl.ds(0, _L)] = jnp.full((_L,), 0, jnp.int32) + ig
            return jnp.where(lane == c, rank, rankvec)

        rankvec = lax.fori_loop(0, _L, _cand,
                                jnp.zeros((_L,), jnp.int32))
        idxvec = lane + grp
        slot = jnp.where(rankvec < _K, rankvec, _K + idxvec)
        myr_v[pl.ds(g * _L, _L)] = slot

    pltpu.async_copy(myv_v, vals_hbm.at[myr_v], sem).wait()
    pltpu.async_copy(myi_v, idxs_hbm.at[myr_v], sem).wait()


def kernel(W, W_id, results, prog_cost, penalty, topN):
    N, B, D = results.shape
    K = _K
    Dc = D // 2
    wr = W.reshape(1, N)
    idr = W_id.reshape(1, N)
    wc = W.reshape(N, 1)
    idc = W_id.reshape(N, 1)
    pen16 = jnp.broadcast_to(penalty, (_L,))

    sc_call = functools.partial(
        pl.kernel,
        mesh=plsc.VectorSubcoreMesh(core_axis_name="c", subcore_axis_name="s"),
        compiler_params=pltpu.CompilerParams(needs_layout_passes=False),
        out_type=[
            jax.ShapeDtypeStruct((_PAD, 128), jnp.float32),
            jax.ShapeDtypeStruct((_PAD, 128), jnp.int32),
        ],
        scratch_types=[
            pltpu.VMEM((N,), jnp.float32),
            pltpu.VMEM((N,), jnp.int32),
            pltpu.VMEM((N,), jnp.float32),
            pltpu.VMEM((_L,), jnp.float32),
            pltpu.VMEM((N,), jnp.float32),
            pltpu.VMEM((_CPW, 128), jnp.float32),
            pltpu.VMEM((_CPW,), jnp.int32),
            pltpu.VMEM((_CPW, 128), jnp.int32),
            pltpu.SemaphoreType.DMA,
        ],
    )(_sc_topk)

    grid = (2, N // _NB)
    out = pl.pallas_call(
        _tc_kernel,
        grid=grid,
        in_specs=[
            pl.BlockSpec((1, N), lambda j, i: (0, 0)),
            pl.BlockSpec((1, N), lambda j, i: (0, 0)),
            pl.BlockSpec((_NB, 1), lambda j, i: (i, 0)),
            pl.BlockSpec((_NB, 1), lambda j, i: (i, 0)),
            pl.BlockSpec((_NB, B, Dc), lambda j, i: (i, 0, j)),
        ],
        out_specs=pl.BlockSpec((B, Dc), lambda j, i: (0, j)),
        out_shape=jax.ShapeDtypeStruct((B, D), jnp.float32),
        compiler_params=pltpu.CompilerParams(
            dimension_semantics=("parallel", "arbitrary"),
        ),
    )(wr, idr, wc, idc, results)
    w2, id2, pc2 = jax.lax.optimization_barrier(
        (W + 0.0, W_id + 0, prog_cost + 0.0))
    neww_pad, topi_pad = sc_call(w2, id2, pc2, pen16)
    return out, neww_pad[:K, 0], topi_pad[:K, 0]


# speedup vs baseline: 1.2858x; 1.2749x over previous
import jax
import jax.numpy as jnp
from jax.experimental import pallas as pl
from jax.experimental.pallas import tpu as pltpu

_NB = 256


def _edge_kernel(wr_ref, idr_ref, pcr_ref, pen_ref, wcb_ref, idcb_ref,
                 res_ref, out_ref, neww_ref, topi_ref):
    j = pl.program_id(0)
    i = pl.program_id(1)

    wr = wr_ref[...]
    idr = idr_ref[...]
    logits_r = jnp.where(idr == 1, wr, -1e30)
    m = jnp.max(logits_r)
    er = jnp.exp(logits_r - m)
    denom = jnp.sum(er)
    ws_r = er / denom

    @pl.when(i == 0)
    def _init():
        out_ref[...] = jnp.zeros_like(out_ref)

    blk = res_ref[...]
    lg_blk = jnp.where(idcb_ref[...] == 1, wcb_ref[...], -1e30)
    w_blk = jnp.exp(lg_blk - m) / denom
    out_ref[...] += jnp.sum(blk * w_blk[:, :, None], axis=0)

    @pl.when(i == 0)
    def _topk():
        n = wr.shape[1]
        pen = pen_ref[0, 0]
        pcr = pcr_ref[...]
        sc_r = ws_r - pen * pcr
        sc_r = jnp.where(idr == 1, sc_r, -jnp.inf)
        sc_c = jnp.transpose(sc_r)

        ii = jax.lax.broadcasted_iota(jnp.int32, (n, n), 0)
        jj = jax.lax.broadcasted_iota(jnp.int32, (n, n), 1)
        beats = (sc_c > sc_r) | ((sc_c == sc_r) & (ii < jj))
        rank = jnp.sum(beats.astype(jnp.int32), axis=0, keepdims=True)

        rows = neww_ref.shape[0]
        r_iota = jax.lax.broadcasted_iota(jnp.int32, (rows, n), 0) + j * rows
        onehot = rank == r_iota
        neww_ref[...] = jnp.sum(jnp.where(onehot, sc_r, 0.0),
                                axis=1, keepdims=True)
        col = jax.lax.broadcasted_iota(jnp.int32, (rows, n), 1)
        topi_ref[...] = jnp.sum(jnp.where(onehot, col, 0),
                                axis=1, keepdims=True)


def kernel(W, W_id, results, prog_cost, penalty, topN):
    N, B, D = results.shape
    K = 256
    Dc = D // 4
    wr = W.reshape(1, N)
    idr = W_id.reshape(1, N)
    pcr = prog_cost.reshape(1, N)
    pen = penalty.reshape(1, 1)
    wc = W.reshape(N, 1)
    idc = W_id.reshape(N, 1)

    grid = (4, N // _NB)
    out, neww, topi = pl.pallas_call(
        _edge_kernel,
        grid=grid,
        in_specs=[
            pl.BlockSpec((1, N), lambda j, i: (0, 0)),
            pl.BlockSpec((1, N), lambda j, i: (0, 0)),
            pl.BlockSpec((1, N), lambda j, i: (0, 0)),
            pl.BlockSpec((1, 1), lambda j, i: (0, 0)),
            pl.BlockSpec((_NB, 1), lambda j, i: (i, 0)),
            pl.BlockSpec((_NB, 1), lambda j, i: (i, 0)),
            pl.BlockSpec((_NB, B, Dc), lambda j, i: (i, 0, j)),
        ],
        out_specs=[
            pl.BlockSpec((B, Dc), lambda j, i: (0, j)),
            pl.BlockSpec((K // 4, 1), lambda j, i: (j, 0)),
            pl.BlockSpec((K // 4, 1), lambda j, i: (j, 0)),
        ],
        out_shape=[
            jax.ShapeDtypeStruct((B, D), jnp.float32),
            jax.ShapeDtypeStruct((K, 1), jnp.float32),
            jax.ShapeDtypeStruct((K, 1), jnp.int32),
        ],
        compiler_params=pltpu.CompilerParams(
            dimension_semantics=("parallel", "arbitrary"),
        ),
    )(wr, idr, pcr, pen, wc, idc, results)
    return out, neww.reshape(K), topi.reshape(K)


# speedup vs baseline: 1.3503x; 1.0502x over previous
import jax
import jax.numpy as jnp
from jax.experimental import pallas as pl
from jax.experimental.pallas import tpu as pltpu

_NB = 256


def _edge_kernel(wr_ref, idr_ref, pcr_ref, pen_ref, wcb_ref, idcb_ref,
                 res_ref, out_ref, neww_ref, topi_ref):
    j = pl.program_id(0)
    i = pl.program_id(1)

    wr = wr_ref[...]
    idr = idr_ref[...]
    logits_r = jnp.where(idr == 1, wr, -1e30)
    m = jnp.max(logits_r)
    er = jnp.exp(logits_r - m)
    denom = jnp.sum(er)
    ws_r = er / denom

    @pl.when(i == 0)
    def _init():
        out_ref[...] = jnp.zeros_like(out_ref)

    blk = res_ref[...]
    lg_blk = jnp.where(idcb_ref[...] == 1, wcb_ref[...], -1e30)
    w_blk = jnp.exp(lg_blk - m) / denom
    out_ref[...] += jnp.sum(blk * w_blk[:, :, None], axis=0)

    @pl.when(i == 0)
    def _topk():
        n = wr.shape[1]
        pen = pen_ref[0, 0]
        pcr = pcr_ref[...]
        sc_r = ws_r - pen * pcr
        sc_r = jnp.where(idr == 1, sc_r, -jnp.inf)
        sc_c = jnp.transpose(sc_r)

        ii = jax.lax.broadcasted_iota(jnp.int32, (n, n), 0)
        jj = jax.lax.broadcasted_iota(jnp.int32, (n, n), 1)
        beats = (sc_c > sc_r) | ((sc_c == sc_r) & (ii < jj))
        rank = jnp.sum(beats.astype(jnp.int32), axis=0, keepdims=True)

        rows = neww_ref.shape[0]
        r_iota = jax.lax.broadcasted_iota(jnp.int32, (rows, n), 0) + j * rows
        onehot = rank == r_iota
        neww_ref[...] = jnp.sum(jnp.where(onehot, sc_r, 0.0),
                                axis=1, keepdims=True)
        col = jax.lax.broadcasted_iota(jnp.int32, (rows, n), 1)
        topi_ref[...] = jnp.sum(jnp.where(onehot, col, 0),
                                axis=1, keepdims=True)


def kernel(W, W_id, results, prog_cost, penalty, topN):
    N, B, D = results.shape
    K = 256
    Dc = D // 2
    wr = W.reshape(1, N)
    idr = W_id.reshape(1, N)
    pcr = prog_cost.reshape(1, N)
    pen = penalty.reshape(1, 1)
    wc = W.reshape(N, 1)
    idc = W_id.reshape(N, 1)

    grid = (2, N // _NB)
    out, neww, topi = pl.pallas_call(
        _edge_kernel,
        grid=grid,
        in_specs=[
            pl.BlockSpec((1, N), lambda j, i: (0, 0)),
            pl.BlockSpec((1, N), lambda j, i: (0, 0)),
            pl.BlockSpec((1, N), lambda j, i: (0, 0)),
            pl.BlockSpec((1, 1), lambda j, i: (0, 0)),
            pl.BlockSpec((_NB, 1), lambda j, i: (i, 0)),
            pl.BlockSpec((_NB, 1), lambda j, i: (i, 0)),
            pl.BlockSpec((_NB, B, Dc), lambda j, i: (i, 0, j)),
        ],
        out_specs=[
            pl.BlockSpec((B, Dc), lambda j, i: (0, j)),
            pl.BlockSpec((K // 2, 1), lambda j, i: (j, 0)),
            pl.BlockSpec((K // 2, 1), lambda j, i: (j, 0)),
        ],
        out_shape=[
            jax.ShapeDtypeStruct((B, D), jnp.float32),
            jax.ShapeDtypeStruct((K, 1), jnp.float32),
            jax.ShapeDtypeStruct((K, 1), jnp.int32),
        ],
        compiler_params=pltpu.CompilerParams(
            dimension_semantics=("arbitrary", "arbitrary"),
        ),
    )(wr, idr, pcr, pen, wc, idc, results)
    return out, neww.reshape(K), topi.reshape(K)


# speedup vs baseline: 1.3761x; 1.0191x over previous
import jax
import jax.numpy as jnp
from jax.experimental import pallas as pl
from jax.experimental.pallas import tpu as pltpu

_NB = 256


def _edge_kernel(wr_ref, idr_ref, pcr_ref, pen_ref, wcb_ref, idcb_ref,
                 res_ref, out_ref, neww_ref, topi_ref):
    j = pl.program_id(0)
    i = pl.program_id(1)

    wr = wr_ref[...]
    idr = idr_ref[...]
    logits_r = jnp.where(idr == 1, wr, -1e30)
    m = jnp.max(logits_r)
    er = jnp.exp(logits_r - m)
    denom = jnp.sum(er)
    ws_r = er / denom

    @pl.when(i == 0)
    def _init():
        out_ref[...] = jnp.zeros_like(out_ref)

    blk = res_ref[...]
    lg_blk = jnp.where(idcb_ref[...] == 1, wcb_ref[...], -1e30)
    w_blk = jnp.exp(lg_blk - m) / denom
    out_ref[...] += jnp.sum(blk * w_blk[:, :, None], axis=0)

    @pl.when(i == 0)
    def _topk():
        n = wr.shape[1]
        pen = pen_ref[0, 0]
        pcr = pcr_ref[...]
        sc_r = ws_r - pen * pcr
        sc_r = jnp.where(idr == 1, sc_r, -jnp.inf)
        sc_c = jnp.transpose(sc_r)

        ii = jax.lax.broadcasted_iota(jnp.int32, (n, n), 0)
        jj = jax.lax.broadcasted_iota(jnp.int32, (n, n), 1)
        beats = (sc_c > sc_r) | ((sc_c == sc_r) & (ii < jj))
        rank = jnp.sum(beats.astype(jnp.int32), axis=0, keepdims=True)

        rows = neww_ref.shape[0]
        r_iota = jax.lax.broadcasted_iota(jnp.int32, (rows, n), 0) + j * rows
        onehot = rank == r_iota
        neww_ref[...] = jnp.sum(jnp.where(onehot, sc_r, 0.0),
                                axis=1, keepdims=True)
        col = jax.lax.broadcasted_iota(jnp.int32, (rows, n), 1)
        topi_ref[...] = jnp.sum(jnp.where(onehot, col, 0),
                                axis=1, keepdims=True)


def kernel(W, W_id, results, prog_cost, penalty, topN):
    N, B, D = results.shape
    K = 256
    Dc = D // 2
    wr = W.reshape(1, N)
    idr = W_id.reshape(1, N)
    pcr = prog_cost.reshape(1, N)
    pen = penalty.reshape(1, 1)
    wc = W.reshape(N, 1)
    idc = W_id.reshape(N, 1)

    grid = (2, N // _NB)
    out, neww, topi = pl.pallas_call(
        _edge_kernel,
        grid=grid,
        in_specs=[
            pl.BlockSpec((1, N), lambda j, i: (0, 0)),
            pl.BlockSpec((1, N), lambda j, i: (0, 0)),
            pl.BlockSpec((1, N), lambda j, i: (0, 0)),
            pl.BlockSpec((1, 1), lambda j, i: (0, 0)),
            pl.BlockSpec((_NB, 1), lambda j, i: (i, 0)),
            pl.BlockSpec((_NB, 1), lambda j, i: (i, 0)),
            pl.BlockSpec((_NB, B, Dc), lambda j, i: (i, 0, j)),
        ],
        out_specs=[
            pl.BlockSpec((B, Dc), lambda j, i: (0, j)),
            pl.BlockSpec((K // 2, 1), lambda j, i: (j, 0)),
            pl.BlockSpec((K // 2, 1), lambda j, i: (j, 0)),
        ],
        out_shape=[
            jax.ShapeDtypeStruct((B, D), jnp.float32),
            jax.ShapeDtypeStruct((K, 1), jnp.float32),
            jax.ShapeDtypeStruct((K, 1), jnp.int32),
        ],
        compiler_params=pltpu.CompilerParams(
            dimension_semantics=("parallel", "arbitrary"),
        ),
    )(wr, idr, pcr, pen, wc, idc, results)
    return out, neww.reshape(K), topi.reshape(K)
